# baseline (speedup 1.0000x reference)
import jax
import jax.numpy as jnp
from jax.experimental import pallas as pl
from jax.experimental.pallas import tpu as pltpu

_N_PER_BLOCK = 16


def _segment_max_block(x_ref, o_ref):
    o_ref[...] = jnp.max(x_ref[...], axis=1)


def kernel(x):
    n, c, h, w = x.shape
    itemsize = jnp.dtype(x.dtype).itemsize

    x_nhwc = jnp.transpose(x, (0, 2, 3, 1)).reshape(n, h * w, c)

    bn = min(_N_PER_BLOCK, n)
    grid = pl.cdiv(n, bn)

    out = pl.pallas_call(
        _segment_max_block,
        out_shape=jax.ShapeDtypeStruct((n, c), x.dtype),
        grid=(grid,),
        in_specs=[pl.BlockSpec((bn, h * w, c), lambda i: (i, 0, 0))],
        out_specs=pl.BlockSpec((bn, c), lambda i: (i, 0)),
        compiler_params=pltpu.CompilerParams(
            dimension_semantics=("parallel",),
            vmem_limit_bytes=48 << 20,
        ),
        cost_estimate=pl.CostEstimate(
            flops=n * c * h * w,
            transcendentals=0,
            bytes_accessed=n * c * h * w * itemsize + n * c * itemsize,
        ),
    )(x_nhwc)

    squeezed = tuple(d for d in (n, c) if d != 1)
    y = out.reshape(squeezed)
    if n == 1:
        y = y[None, ...]
    return y


# speedup vs baseline: 1.0160x; 1.0160x over previous
import jax
import jax.numpy as jnp
from jax.experimental import pallas as pl
from jax.experimental.pallas import tpu as pltpu

_N_PER_BLOCK = 8


def _segment_max_block(x_ref, o_ref):
    o_ref[...] = jnp.max(x_ref[...], axis=1)


def kernel(x):
    n, c, h, w = x.shape
    itemsize = jnp.dtype(x.dtype).itemsize

    x_nhwc = jnp.transpose(x, (0, 2, 3, 1)).reshape(n, h * w, c)

    bn = min(_N_PER_BLOCK, n)
    grid = pl.cdiv(n, bn)

    out = pl.pallas_call(
        _segment_max_block,
        out_shape=jax.ShapeDtypeStruct((n, c), x.dtype),
        grid=(grid,),
        in_specs=[pl.BlockSpec((bn, h * w, c), lambda i: (i, 0, 0))],
        out_specs=pl.BlockSpec((bn, c), lambda i: (i, 0)),
        compiler_params=pltpu.CompilerParams(
            dimension_semantics=("parallel",),
            vmem_limit_bytes=48 << 20,
        ),
        cost_estimate=pl.CostEstimate(
            flops=n * c * h * w,
            transcendentals=0,
            bytes_accessed=n * c * h * w * itemsize + n * c * itemsize,
        ),
    )(x_nhwc)

    squeezed = tuple(d for d in (n, c) if d != 1)
    y = out.reshape(squeezed)
    if n == 1:
        y = y[None, ...]
    return y
